# baseline (speedup 1.0000x reference)
import functools
import math

import jax
import jax.numpy as jnp
from jax.experimental import pallas as pl
from jax.experimental.pallas import tpu as pltpu

_FACTOR = 0.0005


def _one_head(q, ks, k, v, n_top, L_K, L, D):
    S = jax.lax.dot_general(q, ks, (((1,), (1,)), ((), ())),
                            preferred_element_type=jnp.float32)
    M = jnp.max(S, axis=1) - jnp.sum(S, axis=1) / L_K
    R = 128
    Mv0 = M.reshape(L // R, R)
    flat_iota = (jax.lax.broadcasted_iota(jnp.int32, (L // R, R), 0) * R
                 + jax.lax.broadcasted_iota(jnp.int32, (L // R, R), 1))

    def body(i, carry):
        Mv, idxs = carry
        m = jnp.max(Mv)
        idx = jnp.min(jnp.where(Mv == m, flat_iota, L))
        Mv = jnp.where(flat_iota == idx, -jnp.inf, Mv)
        sel_iota = jax.lax.broadcasted_iota(jnp.int32, (n_top,), 0)
        idxs = jnp.where(sel_iota == i, idx, idxs)
        return Mv, idxs

    _, idxs = jax.lax.fori_loop(
        0, n_top, body, (Mv0, jnp.zeros((n_top,), jnp.int32)))

    q_iota = jax.lax.broadcasted_iota(jnp.int32, (n_top, L), 1)
    onehot = (q_iota == idxs[:, None]).astype(jnp.float32)
    sel_q = jnp.dot(onehot, q, preferred_element_type=jnp.float32)
    scores = jax.lax.dot_general(sel_q, k, (((1,), (1,)), ((), ())),
                                 preferred_element_type=jnp.float32)
    scores = scores * (1.0 / math.sqrt(D))
    scores = scores - jnp.max(scores, axis=1, keepdims=True)
    w = jnp.exp(scores)
    w = w / jnp.sum(w, axis=1, keepdims=True)
    ctx = jnp.dot(w, v, preferred_element_type=jnp.float32)

    return jax.lax.dot_general(onehot, ctx, (((0,), (0,)), ((), ())),
                               preferred_element_type=jnp.float32)


def _fused_body(q_ref, ks_ref, k_ref, v_ref, o_ref, *, n_top, L_K, D):
    L = q_ref.shape[1]
    for j in range(q_ref.shape[2] // D):
        sl = slice(j * D, (j + 1) * D)
        o_ref[0, :, sl] = _one_head(
            q_ref[0, :, sl], ks_ref[0, :, sl], k_ref[0, :, sl], v_ref[0, :, sl],
            n_top, L_K, L, D)


def kernel(queries, keys, values):
    B, L, H, D = queries.shape
    L_K = keys.shape[1]
    n_top = max(int(L * _FACTOR * math.log(L_K)), 1)
    U_part = min(n_top, L_K)
    perm = jax.random.permutation(jax.random.key(42), L_K)[:U_part]
    k_sample = keys[:, perm, :, :]

    HP = 128 // D if D < 128 else 1
    q3 = queries.reshape(B, L, H * D)
    k3 = keys.reshape(B, L, H * D)
    v3 = values.reshape(B, L, H * D)
    ks3 = k_sample.reshape(B, U_part, H * D)

    grid = (B, H // HP)
    spec = pl.BlockSpec((1, L, HP * D), lambda b, h: (b, 0, h))
    ksspec = pl.BlockSpec((1, U_part, HP * D), lambda b, h: (b, 0, h))
    body = functools.partial(_fused_body, n_top=n_top, L_K=L_K, D=D)
    out = pl.pallas_call(
        body,
        grid=grid,
        in_specs=[spec, ksspec, spec, spec],
        out_specs=spec,
        out_shape=jax.ShapeDtypeStruct((B, L, H * D), jnp.float32),
        compiler_params=pltpu.CompilerParams(
            dimension_semantics=("parallel", "parallel"),
        ),
    )(q3, ks3, k3, v3)
    return out.reshape(B, L, H, D)


# speedup vs baseline: 1.0908x; 1.0908x over previous
import functools
import math

import jax
import jax.numpy as jnp
from jax.experimental import pallas as pl
from jax.experimental.pallas import tpu as pltpu
from jax.experimental.pallas import tpu_sc as plsc

_FACTOR = 0.0005
_NSEL = 48
_LT = 512


def _a_body(ks_ref, q_ref, mo_ref, *, H, D, L_K):
    for h in range(H):
        ks = ks_ref[0, :, h * D:(h + 1) * D]
        qh = q_ref[0, :, h * D:(h + 1) * D]
        st = jax.lax.dot_general(ks, qh, (((1,), (1,)), ((), ())),
                                 preferred_element_type=jnp.float32)
        mo_ref[0, h, :] = jnp.max(st, axis=0) - jnp.sum(st, axis=0) * (1.0 / L_K)


def _shuf(x, s):
    idx = jax.lax.broadcasted_iota(jnp.int32, (16,), 0) ^ s
    return jax.lax.gather(
        x, idx[:, None],
        jax.lax.GatherDimensionNumbers((), (0,), (0,)), (1,),
        mode=jax.lax.GatherScatterMode.PROMISE_IN_BOUNDS)


def _xmax(x):
    for s in (1, 2, 4, 8):
        x = jnp.maximum(x, _shuf(x, s))
    return x


def _xmin(x):
    for s in (1, 2, 4, 8):
        x = jnp.minimum(x, _shuf(x, s))
    return x


def _sc_body(m_hbm, q_hbm, idx_out, selq_out, m_v, idxb_v, ridx_v, rows_v, sem,
             *, BH, H, L, D, n_top):
    c = jax.lax.axis_index("c")
    s = jax.lax.axis_index("s")
    wid = s * 2 + c

    @pl.when(wid < BH)
    def _():
        pltpu.sync_copy(m_hbm.at[wid], m_v)
        lane = jax.lax.broadcasted_iota(jnp.int32, (16,), 0)
        neg = jnp.full((16,), -jnp.inf, jnp.float32)
        ivs = [jnp.zeros((16,), jnp.int32) for _ in range(3)]
        nrow = L // 128

        for i in range(n_top):
            def scanrow(r, carry):
                bv, bi = carry
                for j in range(8):
                    v = m_v[r, pl.ds(j * 16, 16)]
                    p = v > bv
                    bv = jnp.where(p, v, bv)
                    bi = jnp.where(p, r * 8 + j, bi)
                return bv, bi

            bv, bi = jax.lax.fori_loop(
                0, nrow, scanrow, (neg, jnp.zeros((16,), jnp.int32)))
            bmv = _xmax(bv)
            flatc = jnp.where(bv == bmv, bi * 16 + lane, jnp.int32(1 << 30))
            fminv = _xmin(flatc)
            ivs[i // 16] = jnp.where(lane == (i % 16), fminv, ivs[i // 16])
            fr = fminv[0]
            row = fr // 128
            colc = (fr % 128) // 16 * 16
            cv = m_v[row, pl.ds(colc, 16)]
            m_v[row, pl.ds(colc, 16)] = jnp.where(lane == fr % 16, neg, cv)

        zero16 = jnp.zeros((16,), jnp.int32)
        for r in range(8):
            for j in range(8):
                idxb_v[r, pl.ds(j * 16, 16)] = (
                    ivs[j] if (r == 0 and j < 3) else zero16)
        b = wid // H
        pltpu.sync_copy(idxb_v, idx_out.at[wid])
        for j in range(3):
            ridx_v[pl.ds(j * 16, 16)] = ivs[j] + b * L
        pltpu.async_copy(q_hbm.at[ridx_v], rows_v, sem).wait()
        pltpu.sync_copy(rows_v, selq_out.at[wid])


def _c_body(selq_ref, k_ref, v_ref, ctx_ref, acc, mm, ll, *, H, D, T, scale):
    t = pl.program_id(1)

    @pl.when(t == 0)
    def _init():
        mm[...] = jnp.full(mm.shape, -jnp.inf, jnp.float32)
        ll[...] = jnp.zeros(ll.shape, jnp.float32)
        acc[...] = jnp.zeros(acc.shape, jnp.float32)

    for h in range(H):
        sq = selq_ref[0, h][:, h * D:(h + 1) * D]
        kh = k_ref[0, :, h * D:(h + 1) * D]
        vh = v_ref[0, :, h * D:(h + 1) * D]
        sc = jax.lax.dot_general(sq, kh, (((1,), (1,)), ((), ())),
                                 preferred_element_type=jnp.float32) * scale
        m_old = mm[h][:, 0:1]
        m_new = jnp.maximum(m_old, jnp.max(sc, axis=1, keepdims=True))
        alpha = jnp.exp(m_old - m_new)
        p = jnp.exp(sc - m_new)
        l_new = ll[h][:, 0:1] * alpha + jnp.sum(p, axis=1, keepdims=True)
        acc_new = acc[h] * alpha + jnp.dot(p, vh, preferred_element_type=jnp.float32)
        mm[h] = jnp.broadcast_to(m_new, mm.shape[1:])
        ll[h] = jnp.broadcast_to(l_new, ll.shape[1:])
        acc[h] = acc_new

        @pl.when(t == T - 1)
        def _fin():
            ctx_ref[0, h] = acc_new / l_new


def _d_body(idx_ref, ctx_ref, o_ref, *, H, D, LT, n_top):
    t = pl.program_id(1)
    row_i = jax.lax.broadcasted_iota(jnp.int32, (LT, 128), 0) + t * LT
    valid = jax.lax.broadcasted_iota(jnp.int32, (LT, 128), 1) < n_top
    zpad = jnp.zeros((128 - _NSEL, D), jnp.float32)
    for h in range(H):
        idxb = jnp.broadcast_to(idx_ref[0, h:h + 1, :], (LT, 128))
        oh = jnp.where((idxb == row_i) & valid, 1.0, 0.0)
        ctxp = jnp.concatenate([ctx_ref[0, h], zpad], axis=0)
        o_ref[0, :, h * D:(h + 1) * D] = jnp.dot(
            oh, ctxp, preferred_element_type=jnp.float32)


def _topk_gather_sc(m2, q2, B, H, L, D, n_top):
    BH = B * H
    HD = q2.shape[1]
    body = functools.partial(_sc_body, BH=BH, H=H, L=L, D=D, n_top=n_top)
    f = pl.kernel(
        body,
        out_type=(jax.ShapeDtypeStruct((BH, 8, 128), jnp.int32),
                  jax.ShapeDtypeStruct((BH, _NSEL, HD), jnp.float32)),
        mesh=plsc.VectorSubcoreMesh(core_axis_name="c", subcore_axis_name="s"),
        scratch_types=[
            pltpu.VMEM((L // 128, 128), jnp.float32),
            pltpu.VMEM((8, 128), jnp.int32),
            pltpu.VMEM((_NSEL,), jnp.int32),
            pltpu.VMEM((_NSEL, HD), jnp.float32),
            pltpu.SemaphoreType.DMA,
        ],
    )
    return f(m2.reshape(BH, L // 128, 128), q2)


def kernel(queries, keys, values):
    B, L, H, D = queries.shape
    L_K = keys.shape[1]
    n_top = max(int(L * _FACTOR * math.log(L_K)), 1)
    U = min(n_top, L_K)
    perm = jax.random.permutation(jax.random.key(42), L_K)[:U]
    ks3 = keys[:, perm, :, :].reshape(B, U, H * D)
    q3 = queries.reshape(B, L, H * D)
    k3 = keys.reshape(B, L, H * D)
    v3 = values.reshape(B, L, H * D)
    T = L // _LT
    scale = 1.0 / math.sqrt(D)

    m = pl.pallas_call(
        functools.partial(_a_body, H=H, D=D, L_K=L_K),
        grid=(B, T),
        in_specs=[pl.BlockSpec((1, U, H * D), lambda b, t: (b, 0, 0)),
                  pl.BlockSpec((1, _LT, H * D), lambda b, t: (b, t, 0))],
        out_specs=pl.BlockSpec((1, H, _LT), lambda b, t: (b, 0, t)),
        out_shape=jax.ShapeDtypeStruct((B, H, L), jnp.float32),
        compiler_params=pltpu.CompilerParams(
            dimension_semantics=("parallel", "parallel")),
    )(ks3, q3)

    idx8, selq = _topk_gather_sc(
        m.reshape(B * H, L), queries.reshape(B * L, H * D), B, H, L, D, n_top)
    idx = idx8[:, 0, :].reshape(B, H, 128)

    ctx = pl.pallas_call(
        functools.partial(_c_body, H=H, D=D, T=T, scale=scale),
        grid=(B, T),
        in_specs=[pl.BlockSpec((1, H, _NSEL, H * D), lambda b, t: (b, 0, 0, 0)),
                  pl.BlockSpec((1, _LT, H * D), lambda b, t: (b, t, 0)),
                  pl.BlockSpec((1, _LT, H * D), lambda b, t: (b, t, 0))],
        out_specs=pl.BlockSpec((1, H, _NSEL, D), lambda b, t: (b, 0, 0, 0)),
        out_shape=jax.ShapeDtypeStruct((B, H, _NSEL, D), jnp.float32),
        scratch_shapes=[pltpu.VMEM((H, _NSEL, D), jnp.float32),
                        pltpu.VMEM((H, _NSEL, 128), jnp.float32),
                        pltpu.VMEM((H, _NSEL, 128), jnp.float32)],
        compiler_params=pltpu.CompilerParams(
            dimension_semantics=("arbitrary", "arbitrary")),
    )(selq.reshape(B, H, _NSEL, H * D), k3, v3)

    out3 = pl.pallas_call(
        functools.partial(_d_body, H=H, D=D, LT=_LT, n_top=n_top),
        grid=(B, T),
        in_specs=[pl.BlockSpec((1, H, 128), lambda b, t: (b, 0, 0)),
                  pl.BlockSpec((1, H, _NSEL, D), lambda b, t: (b, 0, 0, 0))],
        out_specs=pl.BlockSpec((1, _LT, H * D), lambda b, t: (b, t, 0)),
        out_shape=jax.ShapeDtypeStruct((B, L, H * D), jnp.float32),
        compiler_params=pltpu.CompilerParams(
            dimension_semantics=("parallel", "parallel")),
    )(idx, ctx)
    return out3.reshape(B, L, H, D)


# speedup vs baseline: 1.1103x; 1.0179x over previous
import functools
import math

import jax
import jax.numpy as jnp
from jax.experimental import pallas as pl
from jax.experimental.pallas import tpu as pltpu
from jax.experimental.pallas import tpu_sc as plsc

_FACTOR = 0.0005
_NSEL = 48
_LT = 512


def _a_body(ks_ref, q_ref, mo_ref, *, D, L_K, L):
    for j in range(2):
        ks = ks_ref[0, :, j * D:(j + 1) * D]
        qh = q_ref[0, :, j * D:(j + 1) * D]
        st = jax.lax.dot_general(ks, qh, (((1,), (1,)), ((), ())),
                                 preferred_element_type=jnp.float32)
        mrow = jnp.max(st, axis=0) - jnp.sum(st, axis=0) * (1.0 / L_K)
        mo_ref[j] = mrow.reshape(L // 128, 128)


def _shuf(x, s):
    idx = jax.lax.broadcasted_iota(jnp.int32, (16,), 0) ^ s
    return jax.lax.gather(
        x, idx[:, None],
        jax.lax.GatherDimensionNumbers((), (0,), (0,)), (1,),
        mode=jax.lax.GatherScatterMode.PROMISE_IN_BOUNDS)


def _xmax(x):
    for s in (1, 2, 4, 8):
        x = jnp.maximum(x, _shuf(x, s))
    return x


def _xmin(x):
    for s in (1, 2, 4, 8):
        x = jnp.minimum(x, _shuf(x, s))
    return x


def _sc_body(m_hbm, q_hbm, idx_out, selq_out, m_v, idxb_v, ridx_v, rows_v, sem,
             *, BH, H, L, D, n_top):
    c = jax.lax.axis_index("c")
    s = jax.lax.axis_index("s")
    wid = s * 2 + c

    @pl.when(wid < BH)
    def _():
        pltpu.sync_copy(m_hbm.at[wid], m_v)
        lane = jax.lax.broadcasted_iota(jnp.int32, (16,), 0)
        neg = jnp.full((16,), -jnp.inf, jnp.float32)
        ivs = [jnp.zeros((16,), jnp.int32) for _ in range(3)]
        nrow = L // 128

        for i in range(n_top):
            def scanrow(r, carry):
                bv, bi = carry
                for j in range(8):
                    v = m_v[r, pl.ds(j * 16, 16)]
                    p = v > bv
                    bv = jnp.where(p, v, bv)
                    bi = jnp.where(p, r * 8 + j, bi)
                return bv, bi

            bv, bi = jax.lax.fori_loop(
                0, nrow, scanrow, (neg, jnp.zeros((16,), jnp.int32)))
            bmv = _xmax(bv)
            flatc = jnp.where(bv == bmv, bi * 16 + lane, jnp.int32(1 << 30))
            fminv = _xmin(flatc)
            ivs[i // 16] = jnp.where(lane == (i % 16), fminv, ivs[i // 16])
            fr = fminv[0]
            row = fr // 128
            colc = (fr % 128) // 16 * 16
            cv = m_v[row, pl.ds(colc, 16)]
            m_v[row, pl.ds(colc, 16)] = jnp.where(lane == fr % 16, neg, cv)

        zero16 = jnp.zeros((16,), jnp.int32)
        for r in range(8):
            for j in range(8):
                idxb_v[r, pl.ds(j * 16, 16)] = (
                    ivs[j] if (r == 0 and j < 3) else zero16)
        b = wid // H
        pltpu.sync_copy(idxb_v, idx_out.at[wid])
        for j in range(3):
            ridx_v[pl.ds(j * 16, 16)] = ivs[j] + b * L
        pltpu.async_copy(q_hbm.at[ridx_v], rows_v, sem).wait()
        pltpu.sync_copy(rows_v, selq_out.at[wid])


def _c_body(selq_ref, k_ref, v_ref, ctx_ref, acc, mm, ll, *, H, D, T, scale):
    t = pl.program_id(1)

    @pl.when(t == 0)
    def _init():
        mm[...] = jnp.full(mm.shape, -jnp.inf, jnp.float32)
        ll[...] = jnp.zeros(ll.shape, jnp.float32)
        acc[...] = jnp.zeros(acc.shape, jnp.float32)

    for h in range(H):
        sq = selq_ref[0, h][:, h * D:(h + 1) * D]
        kh = k_ref[0, :, h * D:(h + 1) * D]
        vh = v_ref[0, :, h * D:(h + 1) * D]
        sc = jax.lax.dot_general(sq, kh, (((1,), (1,)), ((), ())),
                                 preferred_element_type=jnp.float32) * scale
        m_old = mm[h][:, 0:1]
        m_new = jnp.maximum(m_old, jnp.max(sc, axis=1, keepdims=True))
        alpha = jnp.exp(m_old - m_new)
        p = jnp.exp(sc - m_new)
        l_new = ll[h][:, 0:1] * alpha + jnp.sum(p, axis=1, keepdims=True)
        acc_new = acc[h] * alpha + jnp.dot(p, vh, preferred_element_type=jnp.float32)
        mm[h] = jnp.broadcast_to(m_new, mm.shape[1:])
        ll[h] = jnp.broadcast_to(l_new, ll.shape[1:])
        acc[h] = acc_new

        @pl.when(t == T - 1)
        def _fin():
            ctx_ref[0, h] = acc_new / l_new


def _d_body(idx_ref, ctx_ref, o_ref, *, H, D, LT, n_top):
    t = pl.program_id(1)
    row_i = jax.lax.broadcasted_iota(jnp.int32, (LT, 128), 0) + t * LT
    valid = jax.lax.broadcasted_iota(jnp.int32, (LT, 128), 1) < n_top
    zpad = jnp.zeros((128 - _NSEL, D), jnp.float32)
    for h in range(H):
        idxb = jnp.broadcast_to(idx_ref[h, 0:1, :], (LT, 128))
        oh = jnp.where((idxb == row_i) & valid, 1.0, 0.0)
        ctxp = jnp.concatenate([ctx_ref[0, h], zpad], axis=0)
        o_ref[0, :, h * D:(h + 1) * D] = jnp.dot(
            oh, ctxp, preferred_element_type=jnp.float32)


def _topk_gather_sc(m2, q2, B, H, L, D, n_top):
    BH = B * H
    HD = q2.shape[1]
    body = functools.partial(_sc_body, BH=BH, H=H, L=L, D=D, n_top=n_top)
    f = pl.kernel(
        body,
        out_type=(jax.ShapeDtypeStruct((BH, 8, 128), jnp.int32),
                  jax.ShapeDtypeStruct((BH, _NSEL, HD), jnp.float32)),
        mesh=plsc.VectorSubcoreMesh(core_axis_name="c", subcore_axis_name="s"),
        scratch_types=[
            pltpu.VMEM((L // 128, 128), jnp.float32),
            pltpu.VMEM((8, 128), jnp.int32),
            pltpu.VMEM((_NSEL,), jnp.int32),
            pltpu.VMEM((_NSEL, HD), jnp.float32),
            pltpu.SemaphoreType.DMA,
        ],
    )
    return f(m2, q2)


def kernel(queries, keys, values):
    B, L, H, D = queries.shape
    L_K = keys.shape[1]
    n_top = max(int(L * _FACTOR * math.log(L_K)), 1)
    U = min(n_top, L_K)
    perm = jax.random.permutation(jax.random.key(42), L_K)[:U]
    ks3 = keys[:, perm, :, :].reshape(B, U, H * D)
    q3 = queries.reshape(B, L, H * D)
    k3 = keys.reshape(B, L, H * D)
    v3 = values.reshape(B, L, H * D)
    T = L // _LT
    scale = 1.0 / math.sqrt(D)

    m3 = pl.pallas_call(
        functools.partial(_a_body, D=D, L_K=L_K, L=L),
        grid=(B, H // 2),
        in_specs=[pl.BlockSpec((1, U, 2 * D), lambda b, p: (b, 0, p)),
                  pl.BlockSpec((1, L, 2 * D), lambda b, p: (b, 0, p))],
        out_specs=pl.BlockSpec((2, L // 128, 128), lambda b, p: (b * (H // 2) + p, 0, 0)),
        out_shape=jax.ShapeDtypeStruct((B * H, L // 128, 128), jnp.float32),
        compiler_params=pltpu.CompilerParams(
            dimension_semantics=("parallel", "parallel")),
    )(ks3, q3)

    idx8, selq = _topk_gather_sc(
        m3, q3.reshape(B * L, H * D), B, H, L, D, n_top)

    ctx = pl.pallas_call(
        functools.partial(_c_body, H=H, D=D, T=T, scale=scale),
        grid=(B, T),
        in_specs=[pl.BlockSpec((1, H, _NSEL, H * D), lambda b, t: (b, 0, 0, 0)),
                  pl.BlockSpec((1, _LT, H * D), lambda b, t: (b, t, 0)),
                  pl.BlockSpec((1, _LT, H * D), lambda b, t: (b, t, 0))],
        out_specs=pl.BlockSpec((1, H, _NSEL, D), lambda b, t: (b, 0, 0, 0)),
        out_shape=jax.ShapeDtypeStruct((B, H, _NSEL, D), jnp.float32),
        scratch_shapes=[pltpu.VMEM((H, _NSEL, D), jnp.float32),
                        pltpu.VMEM((H, _NSEL, 128), jnp.float32),
                        pltpu.VMEM((H, _NSEL, 128), jnp.float32)],
        compiler_params=pltpu.CompilerParams(
            dimension_semantics=("arbitrary", "arbitrary")),
    )(selq.reshape(B, H, _NSEL, H * D), k3, v3)

    out3 = pl.pallas_call(
        functools.partial(_d_body, H=H, D=D, LT=_LT, n_top=n_top),
        grid=(B, T),
        in_specs=[pl.BlockSpec((H, 8, 128), lambda b, t: (b, 0, 0)),
                  pl.BlockSpec((1, H, _NSEL, D), lambda b, t: (b, 0, 0, 0))],
        out_specs=pl.BlockSpec((1, _LT, H * D), lambda b, t: (b, t, 0)),
        out_shape=jax.ShapeDtypeStruct((B, L, H * D), jnp.float32),
        compiler_params=pltpu.CompilerParams(
            dimension_semantics=("parallel", "parallel")),
    )(idx8, ctx)
    return out3.reshape(B, L, H, D)
